# baseline (speedup 1.0000x reference)
import functools

import jax
import jax.numpy as jnp
from jax import lax
from jax.experimental import pallas as pl
from jax.experimental.pallas import tpu as pltpu
from jax.experimental.pallas import tpu_sc as plsc

_NUM_PATCHES = 4
_PATCH_RADIUS = 0.3


def _vgather(x, idx):
    dn = lax.GatherDimensionNumbers(
        offset_dims=(), collapsed_slice_dims=(0,), start_index_map=(0,))
    return lax.gather(x, idx[:, None], dn, (1,),
                      mode=lax.GatherScatterMode.PROMISE_IN_BOUNDS)


def _fps_mask_body(xyz_ref, mask_ref):
    x = xyz_ref[0]
    n = x.shape[1]
    dist = jnp.full((1, n), 1e10, dtype=jnp.float32)
    far = jnp.int32(0)
    col = lax.broadcasted_iota(jnp.int32, (1, n), 1)
    for step in range(_NUM_PATCHES):
        c = jnp.sum(jnp.where(col == far, x, 0.0), axis=1, keepdims=True)
        d = x - c
        d2 = jnp.sum(d * d, axis=0, keepdims=True)
        dist = jnp.minimum(dist, d2)
        if step < _NUM_PATCHES - 1:
            mx = jnp.max(dist)
            iota = lax.broadcasted_iota(jnp.int32, (1, n), 1)
            cand = jnp.where(dist == mx, iota, n)
            far = jnp.min(cand)
    mask_ref[0] = (jnp.sqrt(dist) >= _PATCH_RADIUS).astype(jnp.int32)


def _fps_mask_call(xyz_t):
    b, _, n = xyz_t.shape
    return pl.pallas_call(
        _fps_mask_body,
        grid=(b,),
        in_specs=[pl.BlockSpec((1, 3, n), lambda i: (i, 0, 0))],
        out_specs=pl.BlockSpec((1, 1, n), lambda i: (i, 0, 0)),
        out_shape=jax.ShapeDtypeStruct((b, 1, n), jnp.int32),
    )(xyz_t).reshape(b, n)


def _make_sc_compact_fill(B, N, C, CP):
    nchunk = N // 16
    nrow = N // 128
    info = plsc.get_sparse_core_info()
    nworkers = info.num_cores * info.num_subcores
    batches_per_worker = B // nworkers
    mesh = plsc.VectorSubcoreMesh(core_axis_name="c", subcore_axis_name="s")

    @functools.partial(
        pl.kernel,
        mesh=mesh,
        compiler_params=pltpu.CompilerParams(
            needs_layout_passes=False, use_tc_tiling_on_sc=False),
        out_type=jax.ShapeDtypeStruct((B * N, C), jnp.float32),
        scratch_types=[
            pltpu.VMEM((N,), jnp.int32),
            pltpu.VMEM((N + 16,), jnp.int32),
            pltpu.VMEM((nrow, 128), jnp.int32),
            pltpu.VMEM((2, 128, CP), jnp.float32),
            pltpu.SemaphoreType.DMA,
            pltpu.SemaphoreType.DMA,
        ],
    )
    def sc_kernel(pts_hbm, mask_hbm, zrow_hbm, out_hbm,
                  mask_v, vidx_v, g_v, rows_v, sem0, sem1):
        wid = lax.axis_index("s") * info.num_cores + lax.axis_index("c")
        for k in range(batches_per_worker):
            b = wid * batches_per_worker + k
            base = b * N
            pltpu.sync_copy(mask_hbm.at[b], mask_v)

            lane = lax.iota(jnp.int32, 16)

            def compact_body(i, off_vec):
                m_i = mask_v[pl.ds(i * 16, 16)]
                keep = m_i > 0
                ivec = lane + i * 16
                s = m_i
                for k in (1, 2, 4, 8):
                    g = _vgather(s, jnp.maximum(lane - k, 0))
                    s = s + jnp.where(lane >= k, g, 0)
                pos = off_vec + s - m_i
                plsc.store_scatter(vidx_v, [pos], ivec, mask=keep)
                return off_vec + plsc.all_reduce_population_count(keep)

            nv_vec = lax.fori_loop(0, nchunk, compact_body,
                                   jnp.zeros((16,), jnp.int32))
            nv = nv_vec[0]

            @pl.when(nv > 0)
            def _fill():
                def gidx_body(i, _):
                    iv = lane + i * 16
                    r = lax.rem(iv, nv_vec)
                    v = plsc.load_gather(vidx_v, [r])
                    g_v[i // 8, pl.ds((i % 8) * 16, 16)] = v + base
                    return 0

                lax.fori_loop(0, nchunk, gidx_body, 0)

                def row_body(t, _):
                    j0 = 2 * t
                    cp0 = pltpu.make_async_copy(
                        pts_hbm.at[g_v.at[j0]], rows_v.at[0], sem0)
                    cp1 = pltpu.make_async_copy(
                        pts_hbm.at[g_v.at[j0 + 1]], rows_v.at[1], sem1)
                    cp0.start()
                    cp1.start()
                    cp0.wait()
                    pltpu.sync_copy(
                        rows_v.at[0].at[:, pl.ds(0, C)],
                        out_hbm.at[pl.ds(base + j0 * 128, 128)])
                    cp1.wait()
                    pltpu.sync_copy(
                        rows_v.at[1].at[:, pl.ds(0, C)],
                        out_hbm.at[pl.ds(base + (j0 + 1) * 128, 128)])
                    return 0

                lax.fori_loop(0, nrow // 2, row_body, 0)

            @pl.when(nv == 0)
            def _zeros():
                pltpu.sync_copy(zrow_hbm, rows_v.at[0].at[:, pl.ds(0, C)])

                def zrow_body(j, _):
                    pltpu.sync_copy(
                        rows_v.at[0].at[:, pl.ds(0, C)],
                        out_hbm.at[pl.ds(base + j * 128, 128)])
                    return 0

                lax.fori_loop(0, nrow, zrow_body, 0)

    return sc_kernel


def kernel(points):
    B, N, C = points.shape
    CP = 8
    xyz_t = jnp.transpose(points[:, :, :3], (0, 2, 1))
    mask = _fps_mask_call(xyz_t)
    pts_pad = jnp.pad(points.reshape(B * N, C), ((0, 0), (0, CP - C)))
    zrow = jnp.zeros((128, C), jnp.float32)
    out_flat = _make_sc_compact_fill(B, N, C, CP)(pts_pad, mask, zrow)
    return out_flat.reshape(B, N, C)


# speedup vs baseline: 1.0454x; 1.0454x over previous
import functools

import jax
import jax.numpy as jnp
from jax import lax
from jax.experimental import pallas as pl
from jax.experimental.pallas import tpu as pltpu
from jax.experimental.pallas import tpu_sc as plsc

_NUM_PATCHES = 4
_PATCH_RADIUS = 0.3


def _vgather(x, idx):
    dn = lax.GatherDimensionNumbers(
        offset_dims=(), collapsed_slice_dims=(0,), start_index_map=(0,))
    return lax.gather(x, idx[:, None], dn, (1,),
                      mode=lax.GatherScatterMode.PROMISE_IN_BOUNDS)


def _fps_mask_body(xyz_ref, mask_ref):
    x = xyz_ref[0]
    _, r, c = x.shape
    n = r * c
    dist = jnp.full((1, r, c), 1e10, dtype=jnp.float32)
    far = jnp.int32(0)
    flat = (lax.broadcasted_iota(jnp.int32, (1, r, c), 1) * c
            + lax.broadcasted_iota(jnp.int32, (1, r, c), 2))
    for step in range(_NUM_PATCHES):
        sel = flat == far
        w = jnp.where(sel, x, 0.0)
        cvec = jnp.sum(jnp.sum(w, axis=2, keepdims=True), axis=1,
                       keepdims=True)
        d = x - cvec
        d2 = jnp.sum(d * d, axis=0, keepdims=True)
        dist = jnp.minimum(dist, d2)
        if step < _NUM_PATCHES - 1:
            mx = jnp.max(dist)
            cand = jnp.where(dist == mx, flat, n)
            far = jnp.min(cand)
    mask_ref[0] = (jnp.sqrt(dist) >= _PATCH_RADIUS).astype(jnp.int32)


def _fps_mask_call(xyz_t):
    b, _, r, c = xyz_t.shape
    return pl.pallas_call(
        _fps_mask_body,
        grid=(b,),
        in_specs=[pl.BlockSpec((1, 3, r, c), lambda i: (i, 0, 0, 0))],
        out_specs=pl.BlockSpec((1, 1, r, c), lambda i: (i, 0, 0, 0)),
        out_shape=jax.ShapeDtypeStruct((b, 1, r, c), jnp.int32),
    )(xyz_t).reshape(b, r * c)


def _make_sc_compact_fill(B, N, C, CP):
    nchunk = N // 16
    nrow = N // 128
    W = 16
    nwave = nrow // W
    info = plsc.get_sparse_core_info()
    nworkers = info.num_cores * info.num_subcores
    bpw = B // nworkers
    mesh = plsc.VectorSubcoreMesh(core_axis_name="c", subcore_axis_name="s")

    @functools.partial(
        pl.kernel,
        mesh=mesh,
        compiler_params=pltpu.CompilerParams(
            needs_layout_passes=False, use_tc_tiling_on_sc=False),
        out_type=jax.ShapeDtypeStruct((B * N, C), jnp.float32),
        scratch_types=[
            pltpu.VMEM((N,), jnp.int32),
            pltpu.VMEM((nrow + 2, 128), jnp.int32),
            pltpu.VMEM((2, W * 128, CP), jnp.float32),
            pltpu.SemaphoreType.DMA,
            pltpu.SemaphoreType.DMA,
        ],
    )
    def sc_kernel(pts_hbm, mask_hbm, zrow_hbm, out_hbm,
                  mask_v, idx_v, wave_v, sem0, sem1):
        wid = lax.axis_index("s") * info.num_cores + lax.axis_index("c")
        lane = lax.iota(jnp.int32, 16)
        sems = (sem0, sem1)
        for k in range(bpw):
            b = wid * bpw + k
            base = b * N
            pltpu.sync_copy(mask_hbm.at[b], mask_v)

            def compact_body(i, off_vec):
                m_i = mask_v[pl.ds(i * 16, 16)]
                keep = m_i > 0
                s = m_i
                for kk in (1, 2, 4, 8):
                    g = _vgather(s, jnp.maximum(lane - kk, 0))
                    s = s + jnp.where(lane >= kk, g, 0)
                pos = off_vec + s - m_i
                plsc.store_scatter(
                    idx_v,
                    [lax.shift_right_logical(pos, 7), pos & 127],
                    base + i * 16 + lane, mask=keep)
                return off_vec + plsc.all_reduce_population_count(keep)

            nv_vec = lax.fori_loop(0, nchunk, compact_body,
                                   jnp.zeros((16,), jnp.int32), unroll=2)
            nv = nv_vec[0]

            @pl.when(nv >= 16)
            def _tail_fast():
                j0 = lax.shift_right_logical(nv, 7)

                def tail_body(i, w_vec):
                    v = plsc.load_gather(
                        idx_v,
                        [lax.shift_right_logical(w_vec, 7), w_vec & 127])
                    idx_v[i // 8, pl.ds((i % 8) * 16, 16)] = v
                    w2 = w_vec + 16
                    return jnp.where(w2 >= nv_vec, w2 - nv_vec, w2)

                w0 = lane + j0 * 128
                w0 = jnp.where(w0 >= nv_vec, w0 - nv_vec, w0)
                lax.fori_loop(j0 * 8, nchunk, tail_body, w0)

            @pl.when((nv > 0) & (nv < 16))
            def _tail_slow():
                def tail_body(i, _):
                    w = lax.rem(lane + i * 16, nv_vec)
                    v = plsc.load_gather(
                        idx_v,
                        [lax.shift_right_logical(w, 7), w & 127])
                    idx_v[i // 8, pl.ds((i % 8) * 16, 16)] = v
                    return 0

                lax.fori_loop(0, nchunk, tail_body, 0)

            @pl.when(nv > 0)
            def _gather():
                def issue(wv, buf):
                    for c in range(W):
                        pltpu.make_async_copy(
                            pts_hbm.at[idx_v.at[wv * W + c]],
                            wave_v.at[buf].at[pl.ds(c * 128, 128)],
                            sems[buf]).start()

                def drain(wv, buf):
                    for c in range(W):
                        pltpu.make_async_copy(
                            pts_hbm.at[idx_v.at[wv * W + c]],
                            wave_v.at[buf].at[pl.ds(c * 128, 128)],
                            sems[buf]).wait()

                def writeback(wv, buf):
                    pltpu.sync_copy(
                        wave_v.at[buf].at[:, pl.ds(0, C)],
                        out_hbm.at[pl.ds(base + wv * W * 128, W * 128)])

                issue(0, 0)

                def wave_body(t, _):
                    issue(2 * t + 1, 1)
                    drain(2 * t, 0)
                    writeback(2 * t, 0)

                    @pl.when(t < nwave // 2 - 1)
                    def _():
                        issue(2 * t + 2, 0)

                    drain(2 * t + 1, 1)
                    writeback(2 * t + 1, 1)
                    return 0

                lax.fori_loop(0, nwave // 2, wave_body, 0)

            @pl.when(nv == 0)
            def _zeros():
                pltpu.sync_copy(zrow_hbm, wave_v.at[0].at[pl.ds(0, 128)]
                                .at[:, pl.ds(0, C)])

                def zrow_body(j, _):
                    pltpu.sync_copy(
                        wave_v.at[0].at[pl.ds(0, 128)].at[:, pl.ds(0, C)],
                        out_hbm.at[pl.ds(base + j * 128, 128)])
                    return 0

                lax.fori_loop(0, nrow, zrow_body, 0)

    return sc_kernel


def kernel(points):
    B, N, C = points.shape
    CP = 8
    xyz_t = jnp.transpose(points[:, :, :3], (0, 2, 1))
    xyz_t = xyz_t.reshape(B, 3, 8, N // 8)
    mask = _fps_mask_call(xyz_t)
    pts_pad = jnp.pad(points.reshape(B * N, C), ((0, 0), (0, CP - C)))
    zrow = jnp.zeros((128, C), jnp.float32)
    out_flat = _make_sc_compact_fill(B, N, C, CP)(pts_pad, mask, zrow)
    return out_flat.reshape(B, N, C)


# speedup vs baseline: 2.5983x; 2.4854x over previous
import functools

import jax
import jax.numpy as jnp
from jax import lax
from jax.experimental import pallas as pl
from jax.experimental.pallas import tpu as pltpu
from jax.experimental.pallas import tpu_sc as plsc

_NUM_PATCHES = 4
_PATCH_RADIUS = 0.3


def _vgather(x, idx):
    dn = lax.GatherDimensionNumbers(
        offset_dims=(), collapsed_slice_dims=(0,), start_index_map=(0,))
    return lax.gather(x, idx[:, None], dn, (1,),
                      mode=lax.GatherScatterMode.PROMISE_IN_BOUNDS)


def _fps_mask_body(xyz_ref, mask_ref):
    x = xyz_ref[0]
    _, r, c = x.shape
    n = r * c
    dist = jnp.full((1, r, c), 1e10, dtype=jnp.float32)
    far = jnp.int32(0)
    flat = (lax.broadcasted_iota(jnp.int32, (1, r, c), 1) * c
            + lax.broadcasted_iota(jnp.int32, (1, r, c), 2))
    for step in range(_NUM_PATCHES):
        sel = flat == far
        w = jnp.where(sel, x, 0.0)
        cvec = jnp.sum(jnp.sum(w, axis=2, keepdims=True), axis=1,
                       keepdims=True)
        d = x - cvec
        d2 = jnp.sum(d * d, axis=0, keepdims=True)
        dist = jnp.minimum(dist, d2)
        if step < _NUM_PATCHES - 1:
            mx = jnp.max(dist)
            cand = jnp.where(dist == mx, flat, n)
            far = jnp.min(cand)
    mask_ref[0] = (jnp.sqrt(dist) >= _PATCH_RADIUS).astype(jnp.int32)


def _fps_mask_call(xyz_t):
    b, _, r, c = xyz_t.shape
    return pl.pallas_call(
        _fps_mask_body,
        grid=(b,),
        in_specs=[pl.BlockSpec((1, 3, r, c), lambda i: (i, 0, 0, 0))],
        out_specs=pl.BlockSpec((1, 1, r, c), lambda i: (i, 0, 0, 0)),
        out_shape=jax.ShapeDtypeStruct((b, 1, r, c), jnp.int32),
    )(xyz_t).reshape(b, r * c)


def _make_sc_compact_fill(B, N, C, CP):
    nchunk = N // 16
    nrow = N // 128
    W = 16
    nwave = nrow // W
    info = plsc.get_sparse_core_info()
    nworkers = info.num_cores * info.num_subcores
    bpw = B // nworkers
    mesh = plsc.VectorSubcoreMesh(core_axis_name="c", subcore_axis_name="s")

    @functools.partial(
        pl.kernel,
        mesh=mesh,
        compiler_params=pltpu.CompilerParams(
            needs_layout_passes=False, use_tc_tiling_on_sc=False),
        out_type=jax.ShapeDtypeStruct((B * N, CP), jnp.float32),
        scratch_types=[
            pltpu.VMEM((N,), jnp.int32),
            pltpu.VMEM((nrow + 2, 128), jnp.int32),
            pltpu.VMEM((2, W * 128, CP), jnp.float32),
            pltpu.SemaphoreType.DMA,
            pltpu.SemaphoreType.DMA,
        ],
    )
    def sc_kernel(pts_hbm, mask_hbm, zrow_hbm, out_hbm,
                  mask_v, idx_v, wave_v, sem0, sem1):
        wid = lax.axis_index("s") * info.num_cores + lax.axis_index("c")
        lane = lax.iota(jnp.int32, 16)
        sems = (sem0, sem1)
        for k in range(bpw):
            b = wid * bpw + k
            base = b * N
            pltpu.sync_copy(mask_hbm.at[b], mask_v)

            def compact_body(i, off_vec):
                m_i = mask_v[pl.ds(i * 16, 16)]
                keep = m_i > 0
                s = m_i
                for kk in (1, 2, 4, 8):
                    g = _vgather(s, jnp.maximum(lane - kk, 0))
                    s = s + jnp.where(lane >= kk, g, 0)
                pos = off_vec + s - m_i
                plsc.store_scatter(
                    idx_v,
                    [lax.shift_right_logical(pos, 7), pos & 127],
                    base + i * 16 + lane, mask=keep)
                return off_vec + plsc.all_reduce_population_count(keep)

            nv_vec = lax.fori_loop(0, nchunk, compact_body,
                                   jnp.zeros((16,), jnp.int32), unroll=2)
            nv = nv_vec[0]

            @pl.when(nv >= 16)
            def _tail_fast():
                j0 = lax.shift_right_logical(nv, 7)

                def tail_body(i, w_vec):
                    v = plsc.load_gather(
                        idx_v,
                        [lax.shift_right_logical(w_vec, 7), w_vec & 127])
                    idx_v[i // 8, pl.ds((i % 8) * 16, 16)] = v
                    w2 = w_vec + 16
                    return jnp.where(w2 >= nv_vec, w2 - nv_vec, w2)

                w0 = lane + j0 * 128
                w0 = jnp.where(w0 >= nv_vec, w0 - nv_vec, w0)
                lax.fori_loop(j0 * 8, nchunk, tail_body, w0)

            @pl.when((nv > 0) & (nv < 16))
            def _tail_slow():
                def tail_body(i, _):
                    w = lax.rem(lane + i * 16, nv_vec)
                    v = plsc.load_gather(
                        idx_v,
                        [lax.shift_right_logical(w, 7), w & 127])
                    idx_v[i // 8, pl.ds((i % 8) * 16, 16)] = v
                    return 0

                lax.fori_loop(0, nchunk, tail_body, 0)

            @pl.when(nv > 0)
            def _gather():
                def issue(wv, buf):
                    for c in range(W):
                        pltpu.make_async_copy(
                            pts_hbm.at[idx_v.at[wv * W + c]],
                            wave_v.at[buf].at[pl.ds(c * 128, 128)],
                            sems[buf]).start()

                def drain(wv, buf):
                    for c in range(W):
                        pltpu.make_async_copy(
                            pts_hbm.at[idx_v.at[wv * W + c]],
                            wave_v.at[buf].at[pl.ds(c * 128, 128)],
                            sems[buf]).wait()

                def writeback(wv, buf):
                    pltpu.sync_copy(
                        wave_v.at[buf],
                        out_hbm.at[pl.ds(base + wv * W * 128, W * 128)])

                issue(0, 0)

                def wave_body(t, _):
                    issue(2 * t + 1, 1)
                    drain(2 * t, 0)
                    writeback(2 * t, 0)

                    @pl.when(t < nwave // 2 - 1)
                    def _():
                        issue(2 * t + 2, 0)

                    drain(2 * t + 1, 1)
                    writeback(2 * t + 1, 1)
                    return 0

                lax.fori_loop(0, nwave // 2, wave_body, 0)

            @pl.when(nv == 0)
            def _zeros():
                pltpu.sync_copy(zrow_hbm, wave_v.at[0].at[pl.ds(0, 128)])

                def zrow_body(j, _):
                    pltpu.sync_copy(
                        wave_v.at[0].at[pl.ds(0, 128)],
                        out_hbm.at[pl.ds(base + j * 128, 128)])
                    return 0

                lax.fori_loop(0, nrow, zrow_body, 0)

    return sc_kernel


def kernel(points):
    B, N, C = points.shape
    CP = 8
    xyz_t = jnp.transpose(points[:, :, :3], (0, 2, 1))
    xyz_t = xyz_t.reshape(B, 3, 8, N // 8)
    mask = _fps_mask_call(xyz_t)
    pts_pad = jnp.pad(points.reshape(B * N, C), ((0, 0), (0, CP - C)))
    zrow = jnp.zeros((128, CP), jnp.float32)
    out_flat = _make_sc_compact_fill(B, N, C, CP)(pts_pad, mask, zrow)
    return out_flat[:, :C].reshape(B, N, C)


# speedup vs baseline: 2.5994x; 1.0004x over previous
import functools

import jax
import jax.numpy as jnp
from jax import lax
from jax.experimental import pallas as pl
from jax.experimental.pallas import tpu as pltpu
from jax.experimental.pallas import tpu_sc as plsc

_NUM_PATCHES = 4
_PATCH_RADIUS = 0.3


def _vgather(x, idx):
    dn = lax.GatherDimensionNumbers(
        offset_dims=(), collapsed_slice_dims=(0,), start_index_map=(0,))
    return lax.gather(x, idx[:, None], dn, (1,),
                      mode=lax.GatherScatterMode.PROMISE_IN_BOUNDS)


def _fps_mask_body(xyz_ref, mask_ref):
    x = xyz_ref[0]
    _, r, c = x.shape
    n = r * c
    dist = jnp.full((1, r, c), 1e10, dtype=jnp.float32)
    far = jnp.int32(0)
    flat = (lax.broadcasted_iota(jnp.int32, (1, r, c), 1) * c
            + lax.broadcasted_iota(jnp.int32, (1, r, c), 2))
    for step in range(_NUM_PATCHES):
        sel = flat == far
        w = jnp.where(sel, x, 0.0)
        cvec = jnp.sum(jnp.sum(w, axis=2, keepdims=True), axis=1,
                       keepdims=True)
        d = x - cvec
        d2 = jnp.sum(d * d, axis=0, keepdims=True)
        dist = jnp.minimum(dist, d2)
        if step < _NUM_PATCHES - 1:
            mx = jnp.max(dist)
            cand = jnp.where(dist == mx, flat, n)
            far = jnp.min(cand)
    mask_ref[0] = (jnp.sqrt(dist) >= _PATCH_RADIUS).astype(jnp.int32)


def _fps_mask_call(xyz_t):
    b, _, r, c = xyz_t.shape
    return pl.pallas_call(
        _fps_mask_body,
        grid=(b,),
        in_specs=[pl.BlockSpec((1, 3, r, c), lambda i: (i, 0, 0, 0))],
        out_specs=pl.BlockSpec((1, 1, r, c), lambda i: (i, 0, 0, 0)),
        out_shape=jax.ShapeDtypeStruct((b, 1, r, c), jnp.int32),
    )(xyz_t).reshape(b, r * c)


def _make_sc_compact_fill(B, N, C, CP):
    nchunk = N // 16
    nrow = N // 128
    W = 16
    nwave = nrow // W
    info = plsc.get_sparse_core_info()
    nworkers = info.num_cores * info.num_subcores
    bpw = B // nworkers
    mesh = plsc.VectorSubcoreMesh(core_axis_name="c", subcore_axis_name="s")

    @functools.partial(
        pl.kernel,
        mesh=mesh,
        compiler_params=pltpu.CompilerParams(
            needs_layout_passes=False, use_tc_tiling_on_sc=False),
        out_type=jax.ShapeDtypeStruct((B * N, CP), jnp.float32),
        scratch_types=[
            pltpu.VMEM((N,), jnp.int32),
            pltpu.VMEM((nrow + 2, 128), jnp.int32),
            pltpu.VMEM((2, W * 128, CP), jnp.float32),
            pltpu.SemaphoreType.DMA,
            pltpu.SemaphoreType.DMA,
        ],
    )
    def sc_kernel(pts_hbm, mask_hbm, zrow_hbm, out_hbm,
                  mask_v, idx_v, wave_v, sem0, sem1):
        wid = lax.axis_index("s") * info.num_cores + lax.axis_index("c")
        lane = lax.iota(jnp.int32, 16)
        sems = (sem0, sem1)
        for k in range(bpw):
            b = wid * bpw + k
            base = b * N
            pltpu.sync_copy(mask_hbm.at[b], mask_v)

            def compact_body(i, off_vec):
                m_i = mask_v[pl.ds(i * 16, 16)]
                keep = m_i > 0
                s = m_i
                for kk in (1, 2, 4, 8):
                    g = _vgather(s, jnp.maximum(lane - kk, 0))
                    s = s + jnp.where(lane >= kk, g, 0)
                pos = off_vec + s - m_i
                plsc.store_scatter(
                    idx_v,
                    [lax.shift_right_logical(pos, 7), pos & 127],
                    i * 16 + lane, mask=keep)
                return off_vec + plsc.all_reduce_population_count(keep)

            nv_vec = lax.fori_loop(0, nchunk, compact_body,
                                   jnp.zeros((16,), jnp.int32), unroll=2)
            nv = nv_vec[0]

            @pl.when(nv >= 16)
            def _tail_fast():
                j0 = lax.shift_right_logical(nv, 7)

                def tail_body(i, w_vec):
                    v = plsc.load_gather(
                        idx_v,
                        [lax.shift_right_logical(w_vec, 7), w_vec & 127])
                    idx_v[i // 8, pl.ds((i % 8) * 16, 16)] = v
                    w2 = w_vec + 16
                    return jnp.where(w2 >= nv_vec, w2 - nv_vec, w2)

                w0 = lane + j0 * 128
                w0 = jnp.where(w0 >= nv_vec, w0 - nv_vec, w0)
                lax.fori_loop(j0 * 8, nchunk, tail_body, w0)

            @pl.when((nv > 0) & (nv < 16))
            def _tail_slow():
                def tail_body(i, _):
                    w = lax.rem(lane + i * 16, nv_vec)
                    v = plsc.load_gather(
                        idx_v,
                        [lax.shift_right_logical(w, 7), w & 127])
                    idx_v[i // 8, pl.ds((i % 8) * 16, 16)] = v
                    return 0

                lax.fori_loop(0, nchunk, tail_body, 0)

            @pl.when(nv > 0)
            def _gather():
                def issue(wv, buf):
                    for c in range(W):
                        pltpu.make_async_copy(
                            pts_hbm.at[b].at[idx_v.at[wv * W + c]],
                            wave_v.at[buf].at[pl.ds(c * 128, 128)],
                            sems[buf]).start()

                def drain(wv, buf):
                    for c in range(W):
                        pltpu.make_async_copy(
                            pts_hbm.at[b].at[idx_v.at[wv * W + c]],
                            wave_v.at[buf].at[pl.ds(c * 128, 128)],
                            sems[buf]).wait()

                def writeback(wv, buf):
                    pltpu.sync_copy(
                        wave_v.at[buf],
                        out_hbm.at[pl.ds(base + wv * W * 128, W * 128)])

                issue(0, 0)

                def wave_body(t, _):
                    issue(2 * t + 1, 1)
                    drain(2 * t, 0)
                    writeback(2 * t, 0)

                    @pl.when(t < nwave // 2 - 1)
                    def _():
                        issue(2 * t + 2, 0)

                    drain(2 * t + 1, 1)
                    writeback(2 * t + 1, 1)
                    return 0

                lax.fori_loop(0, nwave // 2, wave_body, 0)

            @pl.when(nv == 0)
            def _zeros():
                pltpu.sync_copy(zrow_hbm, wave_v.at[0].at[pl.ds(0, 128)])

                def zrow_body(j, _):
                    pltpu.sync_copy(
                        wave_v.at[0].at[pl.ds(0, 128)],
                        out_hbm.at[pl.ds(base + j * 128, 128)])
                    return 0

                lax.fori_loop(0, nrow, zrow_body, 0)

    return sc_kernel


def kernel(points):
    B, N, C = points.shape
    CP = 8
    xyz_t = jnp.transpose(points[:, :, :3], (0, 2, 1))
    xyz_t = xyz_t.reshape(B, 3, 8, N // 8)
    mask = _fps_mask_call(xyz_t)
    pts_pad = jnp.pad(points, ((0, 0), (0, 0), (0, CP - C)))
    zrow = jnp.zeros((128, CP), jnp.float32)
    out_flat = _make_sc_compact_fill(B, N, C, CP)(pts_pad, mask, zrow)
    return out_flat[:, :C].reshape(B, N, C)
